# baseline (speedup 1.0000x reference)
import functools

import jax
import jax.numpy as jnp
from jax import lax
from jax.experimental import pallas as pl
from jax.experimental.pallas import tpu as pltpu
from jax.experimental.pallas import tpu_sc as plsc

NC = 2
NS = 16
NW = NC * NS
K = 128
W = 16


SUB = 8
C = SUB * K


def _sc_segment_sum(NP, EPW):
    RPT = NP // NS
    KI = EPW // C
    KI2 = KI // 2
    mesh = plsc.VectorSubcoreMesh(
        core_axis_name="c", subcore_axis_name="s",
        num_cores=NC, num_subcores=NS)

    @functools.partial(
        pl.kernel,
        out_type=jax.ShapeDtypeStruct((NC, NP, W), jnp.float32),
        mesh=mesh,
        scratch_types=[
            pltpu.VMEM((2 * SUB, K), jnp.int32),
            pltpu.VMEM((2 * SUB, K), jnp.int32),
            pltpu.VMEM((SUB, K, W), jnp.float32),
            pltpu.VMEM((SUB, K, W), jnp.float32),
            pltpu.VMEM((RPT // 4, W), jnp.float32),
            pltpu.VMEM_SHARED((NP, W), jnp.float32),
            pltpu.SemaphoreType.DMA,
            pltpu.SemaphoreType.DMA,
            pltpu.SemaphoreType.DMA,
            pltpu.SemaphoreType.DMA,
        ],
        compiler_params=pltpu.CompilerParams(use_tc_tiling_on_sc=False),
    )
    def k(table_hbm, idx_hbm, out_hbm,
          idx_v0, idx_v1, rows_v0, rows_v1, stage_v, acc_sh,
          sem_g0, sem_g1, sem_s0, sem_s1):
        c = lax.axis_index("c")
        s = lax.axis_index("s")
        wid = c * NS + s
        row0 = s * RPT
        zrow = jnp.zeros((16,), jnp.float32)

        SR = RPT // 4

        def zero_body(i, carry):
            stage_v[i, :] = zrow
            return carry

        lax.fori_loop(0, SR, zero_body, 0)
        for z in range(4):
            pltpu.sync_copy(stage_v, acc_sh.at[pl.ds(row0 + z * SR, SR)])
        plsc.subcore_barrier()

        def load_idx(blk, idx_v):
            r0 = (wid * KI + blk) * 2 * SUB
            pltpu.sync_copy(idx_hbm.at[pl.ds(r0, 2 * SUB)], idx_v)

        def fire_g(idx_v, rows_v, sem):
            for j in range(SUB):
                pltpu.async_copy(table_hbm.at[idx_v.at[j]],
                                 rows_v.at[j], sem)

        def drain_g(idx_v, rows_v, sem):
            for j in range(SUB):
                pltpu.make_async_copy(table_hbm.at[idx_v.at[j]],
                                      rows_v.at[j], sem).wait()

        def fire_s(idx_v, rows_v, sem):
            for j in range(SUB):
                pltpu.async_copy(rows_v.at[j],
                                 acc_sh.at[idx_v.at[SUB + j]], sem, add=True)

        def drain_s(idx_v, rows_v, sem):
            for j in range(SUB):
                pltpu.make_async_copy(rows_v.at[j],
                                      acc_sh.at[idx_v.at[SUB + j]],
                                      sem).wait()

        load_idx(0, idx_v0)
        fire_g(idx_v0, rows_v0, sem_g0)
        load_idx(1, idx_v1)
        fire_g(idx_v1, rows_v1, sem_g1)
        drain_g(idx_v0, rows_v0, sem_g0)
        fire_s(idx_v0, rows_v0, sem_s0)
        drain_s(idx_v0, rows_v0, sem_s0)
        load_idx(2, idx_v0)
        fire_g(idx_v0, rows_v0, sem_g0)
        drain_g(idx_v1, rows_v1, sem_g1)
        fire_s(idx_v1, rows_v1, sem_s1)

        def body(i, carry):
            drain_s(idx_v1, rows_v1, sem_s1)
            load_idx(2 * i + 1, idx_v1)
            fire_g(idx_v1, rows_v1, sem_g1)
            drain_g(idx_v0, rows_v0, sem_g0)
            fire_s(idx_v0, rows_v0, sem_s0)
            drain_s(idx_v0, rows_v0, sem_s0)
            load_idx(2 * i + 2, idx_v0)
            fire_g(idx_v0, rows_v0, sem_g0)
            drain_g(idx_v1, rows_v1, sem_g1)
            fire_s(idx_v1, rows_v1, sem_s1)
            return carry

        lax.fori_loop(1, KI2, body, 0)
        drain_s(idx_v1, rows_v1, sem_s1)
        drain_g(idx_v0, rows_v0, sem_g0)
        plsc.subcore_barrier()

        for z in range(4):
            pltpu.sync_copy(acc_sh.at[pl.ds(row0 + z * SR, SR)], stage_v)
            pltpu.sync_copy(stage_v, out_hbm.at[c, pl.ds(row0 + z * SR, SR)])

    return k


def _tc_layer1(NP, R):
    grid = NP // R

    def body(a0, a1, t1, w1, b1, s1, tt1, out):
        S = a0[...] + a1[...]
        pos = t1[:, 1:4]
        deg = S[:, 4:5]
        inv = 1.0 / jnp.maximum(deg, 1.0)
        g = (deg > 0).astype(jnp.float32)
        w = w1[...]
        z = lax.dot_general(S[:, 0:4], w, (((1,), (1,)), ((), ())),
                            preferred_element_type=jnp.float32, precision=lax.Precision.HIGHEST)
        q = b1[...] - lax.dot_general(pos, w[:, 1:4], (((1,), (1,)), ((), ())),
                                      preferred_element_type=jnp.float32, precision=lax.Precision.HIGHEST)
        agg = z * inv + g * q
        out[...] = jnp.maximum(agg * s1[...] + tt1[...], 0.0)

    return pl.pallas_call(
        body,
        grid=(grid,),
        in_specs=[
            pl.BlockSpec((R, W), lambda i: (i, 0)),
            pl.BlockSpec((R, W), lambda i: (i, 0)),
            pl.BlockSpec((R, W), lambda i: (i, 0)),
            pl.BlockSpec((16, 4), lambda i: (0, 0)),
            pl.BlockSpec((1, 16), lambda i: (0, 0)),
            pl.BlockSpec((1, 16), lambda i: (0, 0)),
            pl.BlockSpec((1, 16), lambda i: (0, 0)),
        ],
        out_specs=pl.BlockSpec((R, 16), lambda i: (i, 0)),
        out_shape=jax.ShapeDtypeStruct((NP, 16), jnp.float32),
    )


def _tc_layer2(NP, R):
    grid = NP // R

    def body(c0, c1, a0, a1, t1, w2, b2, s2, tt2, out):
        S1 = c0[...] + c1[...]
        A = a0[...] + a1[...]
        pos = t1[:, 1:4]
        Sp = A[:, 1:4]
        deg = A[:, 4:5]
        inv = 1.0 / jnp.maximum(deg, 1.0)
        g = (deg > 0).astype(jnp.float32)
        w = w2[...]
        wf = w[:, 0:16]
        wp = w[:, 16:19]
        z = (lax.dot_general(S1, wf, (((1,), (1,)), ((), ())),
                             preferred_element_type=jnp.float32, precision=lax.Precision.HIGHEST)
             + lax.dot_general(Sp, wp, (((1,), (1,)), ((), ())),
                               preferred_element_type=jnp.float32, precision=lax.Precision.HIGHEST))
        q = b2[...] - lax.dot_general(pos, wp, (((1,), (1,)), ((), ())),
                                      preferred_element_type=jnp.float32, precision=lax.Precision.HIGHEST)
        agg = z * inv + g * q
        out[...] = jnp.maximum(agg * s2[...] + tt2[...], 0.0)

    return pl.pallas_call(
        body,
        grid=(grid,),
        in_specs=[
            pl.BlockSpec((R, W), lambda i: (i, 0)),
            pl.BlockSpec((R, W), lambda i: (i, 0)),
            pl.BlockSpec((R, W), lambda i: (i, 0)),
            pl.BlockSpec((R, W), lambda i: (i, 0)),
            pl.BlockSpec((R, W), lambda i: (i, 0)),
            pl.BlockSpec((32, 19), lambda i: (0, 0)),
            pl.BlockSpec((1, 32), lambda i: (0, 0)),
            pl.BlockSpec((1, 32), lambda i: (0, 0)),
            pl.BlockSpec((1, 32), lambda i: (0, 0)),
        ],
        out_specs=pl.BlockSpec((R, 32), lambda i: (i, 0)),
        out_shape=jax.ShapeDtypeStruct((NP, 32), jnp.float32),
    )


def kernel(x, pos, edge_index, W1, b1, s1, t1, W2, b2, s2, t2):
    N = x.shape[0]
    E = edge_index.shape[1]
    NP = ((N + 16 * K - 1) // (16 * K)) * (16 * K)
    EPW = ((E + 2 * NW * C - 1) // (2 * NW * C)) * 2 * C
    EP = EPW * NW
    KI = EPW // C

    table1 = jnp.concatenate(
        [x, pos, jnp.ones((N, 1), jnp.float32),
         jnp.zeros((N, W - 5), jnp.float32)], axis=1)
    table1 = jnp.pad(table1, ((0, NP - N), (0, 0)))

    srcr = jnp.concatenate(
        [edge_index[0], jnp.full((EP - E,), N, jnp.int32)]
    ).reshape(NW * KI, SUB, K)
    dstr = jnp.concatenate(
        [edge_index[1], jnp.full((EP - E,), N, jnp.int32)]
    ).reshape(NW * KI, SUB, K)
    idx2 = jnp.concatenate([srcr, dstr], axis=1).reshape(-1, K)
    idx2 = jnp.concatenate([idx2, jnp.full((2 * SUB, K), N, jnp.int32)])

    sc_pass = _sc_segment_sum(NP, EPW)
    partA = sc_pass(table1, idx2)

    R = NP // 16
    b1r, s1r, t1r = b1.reshape(1, 16), s1.reshape(1, 16), t1.reshape(1, 16)
    out1 = _tc_layer1(NP, R)(partA[0], partA[1], table1, W1, b1r, s1r, t1r)

    partC = sc_pass(out1, idx2)

    b2r, s2r, t2r = b2.reshape(1, 32), s2.reshape(1, 32), t2.reshape(1, 32)
    out2 = _tc_layer2(NP, R)(partC[0], partC[1], partA[0], partA[1],
                             table1, W2, b2r, s2r, t2r)
    return out2[:N]
